# baseline (speedup 1.0000x reference)
import jax
import jax.numpy as jnp
from jax.experimental import pallas as pl
from jax.experimental.pallas import tpu as pltpu

_TOT = 32 * 512 * 512
_COLS = 512
_ROWS = _TOT // _COLS
_BLKR = 512
_GRID = _ROWS // _BLKR
_CHUNKS = _BLKR // 8


def _tc_body(src_ref, tar_ref, out_ref, acc_ref):
    i = pl.program_id(0)

    @pl.when(i == 0)
    def _():
        acc_ref[...] = jnp.zeros_like(acc_ref)

    asq = acc_ref[0]
    acn = acc_ref[1]
    for k in range(_CHUNKS):
        s = src_ref[k * 8:(k + 1) * 8, :]
        t = tar_ref[k * 8:(k + 1) * 8, :]
        mask = t > 0.05
        d = s - t
        asq = asq + jnp.where(mask, d * d, 0.0)
        acn = acn + jnp.where(mask, 1.0, 0.0)
    acc_ref[0] = asq
    acc_ref[1] = acn

    @pl.when(i == _GRID - 1)
    def _():
        out_ref[0, 0] = jnp.sum(acc_ref[0]) / jnp.sum(acc_ref[1])


def kernel(src, tar):
    src2 = src.reshape(_ROWS, _COLS)
    tar2 = tar.reshape(_ROWS, _COLS)
    out = pl.pallas_call(
        _tc_body,
        grid=(_GRID,),
        in_specs=[
            pl.BlockSpec((_BLKR, _COLS), lambda i: (i, 0)),
            pl.BlockSpec((_BLKR, _COLS), lambda i: (i, 0)),
        ],
        out_specs=pl.BlockSpec(memory_space=pltpu.SMEM),
        out_shape=jax.ShapeDtypeStruct((1, 1), jnp.float32),
        scratch_shapes=[pltpu.VMEM((2, 8, _COLS), jnp.float32)],
    )(src2, tar2)
    return out[0, 0]


# speedup vs baseline: 1.4922x; 1.4922x over previous
import jax
import jax.numpy as jnp
from jax.experimental import pallas as pl
from jax.experimental.pallas import tpu as pltpu

_TOT = 32 * 512 * 512
_COLS = 512
_ROWS = _TOT // _COLS
_BLKR = 2048
_GRID = _ROWS // _BLKR
_CHUNKS = _BLKR // 8


def _tc_body(src_ref, tar_ref, out_ref, acc_ref):
    i = pl.program_id(0)

    @pl.when(i == 0)
    def _():
        acc_ref[...] = jnp.zeros_like(acc_ref)

    asq = acc_ref[0]
    acn = acc_ref[1]
    for k in range(_CHUNKS):
        s = src_ref[k * 8:(k + 1) * 8, :]
        t = tar_ref[k * 8:(k + 1) * 8, :]
        mask = t > 0.05
        d = s - t
        asq = asq + jnp.where(mask, d * d, 0.0)
        acn = acn + jnp.where(mask, 1.0, 0.0)
    acc_ref[0] = asq
    acc_ref[1] = acn

    @pl.when(i == _GRID - 1)
    def _():
        out_ref[0, 0] = jnp.sum(acc_ref[0]) / jnp.sum(acc_ref[1])


def kernel(src, tar):
    src2 = src.reshape(_ROWS, _COLS)
    tar2 = tar.reshape(_ROWS, _COLS)
    out = pl.pallas_call(
        _tc_body,
        grid=(_GRID,),
        in_specs=[
            pl.BlockSpec((_BLKR, _COLS), lambda i: (i, 0)),
            pl.BlockSpec((_BLKR, _COLS), lambda i: (i, 0)),
        ],
        out_specs=pl.BlockSpec(memory_space=pltpu.SMEM),
        out_shape=jax.ShapeDtypeStruct((1, 1), jnp.float32),
        scratch_shapes=[pltpu.VMEM((2, 8, _COLS), jnp.float32)],
    )(src2, tar2)
    return out[0, 0]
